# baseline (speedup 1.0000x reference)
import functools

import jax
import jax.numpy as jnp
from jax import lax
from jax.experimental import pallas as pl
from jax.experimental.pallas import tpu as pltpu
from jax.experimental.pallas import tpu_sc as plsc

P = 16
Q = 32
S = 2048
D = 256
TOTAL_Q = P * Q

L = 16
NW = 32
QB = TOTAL_Q // NW

SCHUNK = 512
NSBLK = S // SCHUNK


def _costs_body(problems_ref, valid_ref, w_ref, costs_ref):
    c = jnp.dot(problems_ref[...], w_ref[...], preferred_element_type=jnp.float32)
    vf = valid_ref[...].astype(jnp.float32)
    costs_ref[...] = c * vf.reshape(P, 1)


HQ = QB // 2


def _sc_reduce_body(costs_hbm, q_hbm, out_hbm, costs_v, q_v0, q_v1, out_v,
                    sem0, sem1):
    wid = lax.axis_index("s") * 2 + lax.axis_index("c")
    qbase = wid * QB
    prob = wid // 2

    cp0 = pltpu.async_copy(q_hbm.at[pl.ds(qbase * S, HQ * S)], q_v0, sem0)
    cp1 = pltpu.async_copy(q_hbm.at[pl.ds((qbase + HQ) * S, HQ * S)], q_v1,
                           sem1)
    pltpu.sync_copy(costs_hbm.at[prob], costs_v)

    zero = jnp.zeros((L,), jnp.float32)

    def half(q_v):
        def body(c, accs):
            c0 = c * (2 * L)
            cc0 = costs_v[pl.ds(c0, L)]
            cc1 = costs_v[pl.ds(c0 + L, L)]
            return tuple(
                accs[i]
                + q_v[pl.ds(i * S + c0, L)] * cc0
                + q_v[pl.ds(i * S + c0 + L, L)] * cc1
                for i in range(HQ))
        return lax.fori_loop(0, S // (2 * L), body,
                             tuple(zero for _ in range(HQ)))

    cp0.wait()
    accs0 = half(q_v0)
    cp1.wait()
    accs1 = half(q_v1)

    lanes = lax.iota(jnp.int32, L)
    tot = zero
    for i, acc in enumerate(accs0 + accs1):
        tot = jnp.where(lanes == i, jnp.sum(acc), tot)
    out_v[...] = tot
    pltpu.sync_copy(out_v, out_hbm.at[pl.ds(qbase, QB)])


_sc_reduce = functools.partial(
    pl.kernel,
    out_type=jax.ShapeDtypeStruct((TOTAL_Q,), jnp.float32),
    mesh=plsc.VectorSubcoreMesh(core_axis_name="c", subcore_axis_name="s"),
    compiler_params=pltpu.CompilerParams(needs_layout_passes=False),
    scratch_types=[
        pltpu.VMEM((S,), jnp.float32),
        pltpu.VMEM((HQ * S,), jnp.float32),
        pltpu.VMEM((HQ * S,), jnp.float32),
        pltpu.VMEM((L,), jnp.float32),
        pltpu.SemaphoreType.DMA,
        pltpu.SemaphoreType.DMA,
    ],
)(_sc_reduce_body)


def kernel(problems, questions_flat_values, questions_outer_row_splits,
           questions_inner_row_splits, valid, W):
    costs = pl.pallas_call(
        _costs_body,
        grid=(NSBLK,),
        in_specs=[
            pl.BlockSpec((P, D), lambda i: (0, 0)),
            pl.BlockSpec((P,), lambda i: (0,)),
            pl.BlockSpec((D, SCHUNK), lambda i: (0, i)),
        ],
        out_specs=pl.BlockSpec((P, SCHUNK), lambda i: (0, i)),
        out_shape=jax.ShapeDtypeStruct((P, S), jnp.float32),
    )(problems, valid, W)
    return _sc_reduce(costs, questions_flat_values)


# speedup vs baseline: 1.0536x; 1.0536x over previous
import functools

import jax
import jax.numpy as jnp
from jax import lax
from jax.experimental import pallas as pl
from jax.experimental.pallas import tpu as pltpu
from jax.experimental.pallas import tpu_sc as plsc

P = 16
Q = 32
S = 2048
D = 256
TOTAL_Q = P * Q

L = 16
NW = 32
QB = TOTAL_Q // NW

SCHUNK = 1024
NSBLK = S // SCHUNK


def _costs_body(problems_ref, valid_ref, w_ref, costs_ref):
    c = jnp.dot(problems_ref[...], w_ref[...], preferred_element_type=jnp.float32)
    vf = (valid_ref[...] != 0).astype(jnp.float32)
    costs_ref[...] = c * vf.reshape(P, 1)


HQ = QB // 2


def _sc_reduce_body(costs_hbm, q_hbm, out_hbm, costs_v, q_v0, q_v1, out_v,
                    sem0, sem1):
    wid = lax.axis_index("s") * 2 + lax.axis_index("c")
    qbase = wid * QB
    prob = wid // 2

    cp0 = pltpu.async_copy(q_hbm.at[pl.ds(qbase * S, HQ * S)], q_v0, sem0)
    cp1 = pltpu.async_copy(q_hbm.at[pl.ds((qbase + HQ) * S, HQ * S)], q_v1,
                           sem1)
    pltpu.sync_copy(costs_hbm.at[prob], costs_v)

    zero = jnp.zeros((L,), jnp.float32)

    def half(q_v):
        def body(c, accs):
            c0 = c * (2 * L)
            cc0 = costs_v[pl.ds(c0, L)]
            cc1 = costs_v[pl.ds(c0 + L, L)]
            return tuple(
                accs[i]
                + q_v[pl.ds(i * S + c0, L)] * cc0
                + q_v[pl.ds(i * S + c0 + L, L)] * cc1
                for i in range(HQ))
        return lax.fori_loop(0, S // (2 * L), body,
                             tuple(zero for _ in range(HQ)))

    cp0.wait()
    accs0 = half(q_v0)
    cp1.wait()
    accs1 = half(q_v1)

    lanes = lax.iota(jnp.int32, L)
    tot = zero
    for i, acc in enumerate(accs0 + accs1):
        tot = jnp.where(lanes == i, jnp.sum(acc), tot)
    out_v[...] = tot
    pltpu.sync_copy(out_v, out_hbm.at[pl.ds(qbase, QB)])


_sc_reduce = functools.partial(
    pl.kernel,
    out_type=jax.ShapeDtypeStruct((TOTAL_Q,), jnp.float32),
    mesh=plsc.VectorSubcoreMesh(core_axis_name="c", subcore_axis_name="s"),
    compiler_params=pltpu.CompilerParams(needs_layout_passes=False),
    scratch_types=[
        pltpu.VMEM((S,), jnp.float32),
        pltpu.VMEM((HQ * S,), jnp.float32),
        pltpu.VMEM((HQ * S,), jnp.float32),
        pltpu.VMEM((L,), jnp.float32),
        pltpu.SemaphoreType.DMA,
        pltpu.SemaphoreType.DMA,
    ],
)(_sc_reduce_body)


def kernel(problems, questions_flat_values, questions_outer_row_splits,
           questions_inner_row_splits, valid, W):
    valid_i8 = valid.view(jnp.int8)
    costs = pl.pallas_call(
        _costs_body,
        grid=(NSBLK,),
        in_specs=[
            pl.BlockSpec((P, D), lambda i: (0, 0)),
            pl.BlockSpec((P,), lambda i: (0,)),
            pl.BlockSpec((D, SCHUNK), lambda i: (0, i)),
        ],
        out_specs=pl.BlockSpec((P, SCHUNK), lambda i: (0, i)),
        out_shape=jax.ShapeDtypeStruct((P, S), jnp.float32),
    )(problems, valid_i8, W)
    return _sc_reduce(costs, questions_flat_values)
